# baseline (speedup 1.0000x reference)
import functools

import jax
import jax.numpy as jnp
from jax import lax
from jax.experimental import pallas as pl
from jax.experimental.pallas import tpu as pltpu
from jax.experimental.pallas import tpu_sc as plsc

N = 10000
E = 320000
D = 128

NC = 2
NS = 16
NW = NC * NS
EPT = E // NW
K = 100
NCH = EPT // K
NPAIR = NCH // 2
NPAD = 10240
RPT = NPAD // NS
DEG_W = 16

_mesh = plsc.VectorSubcoreMesh(core_axis_name="c", subcore_axis_name="s")


@functools.partial(
    pl.kernel,
    out_type=jax.ShapeDtypeStruct((NC, NPAD, DEG_W), jnp.float32),
    mesh=_mesh,
    scratch_types=[
        pltpu.VMEM((NCH, K), jnp.int32),
        pltpu.VMEM((K, DEG_W), jnp.float32),
        pltpu.SemaphoreType.DMA,
        pltpu.VMEM_SHARED((NPAD, DEG_W), jnp.float32),
    ],
)
def _deg_kernel(dst_hbm, z_hbm, out_hbm, dst_v, ones_v, sem, deg_sh):
    c_id = lax.axis_index("c")
    s_id = lax.axis_index("s")
    wid = c_id * NS + s_id

    pltpu.sync_copy(dst_hbm.at[wid], dst_v)

    @pl.when(s_id == 0)
    def _():
        pltpu.sync_copy(z_hbm, deg_sh)

    def fill(i, carry):
        ones_v[i] = jnp.ones((DEG_W,), jnp.float32)
        return carry

    lax.fori_loop(0, K, fill, 0)
    plsc.subcore_barrier()

    def step(i, carry):
        pltpu.async_copy(ones_v, deg_sh.at[dst_v.at[i]], sem, add=True).wait()
        return carry

    lax.fori_loop(0, NCH, step, 0)
    plsc.subcore_barrier()

    pltpu.sync_copy(
        deg_sh.at[pl.ds(s_id * RPT, RPT)],
        out_hbm.at[c_id, pl.ds(s_id * RPT, RPT)],
    )


KA = 100
EPTP = 10400
NCHA = EPTP // KA
NBLKA = 4
BCA = NCHA // NBLKA
PAIRA = BCA // 2


@functools.partial(
    pl.kernel,
    out_type=jax.ShapeDtypeStruct((NC, NPAD, D), jnp.float32),
    mesh=_mesh,
    scratch_types=[
        pltpu.VMEM((BCA, KA), jnp.int32),
        pltpu.VMEM((BCA, KA), jnp.int32),
        pltpu.VMEM((BCA, KA), jnp.int32),
        pltpu.VMEM((BCA, KA), jnp.int32),
        pltpu.VMEM((KA, D), jnp.float32),
        pltpu.VMEM((KA, D), jnp.float32),
        pltpu.SemaphoreType.DMA,
        pltpu.SemaphoreType.DMA,
        pltpu.SemaphoreType.DMA,
        pltpu.SemaphoreType.DMA,
        pltpu.SemaphoreType.DMA,
        pltpu.VMEM_SHARED((NPAD, D), jnp.float32),
    ],
)
def _agg_kernel(h_hbm, src_hbm, dst_hbm, z_hbm, out_hbm,
                src_b0, src_b1, dst_b0, dst_b1, rows0, rows1,
                gsem0, gsem1, ssem0, ssem1, isem, agg_sh):
    src_bufs = (src_b0, src_b1)
    dst_bufs = (dst_b0, dst_b1)
    ring = ((rows0, gsem0, ssem0), (rows1, gsem1, ssem1))
    c_id = lax.axis_index("c")
    s_id = lax.axis_index("s")
    wid = c_id * NS + s_id

    pltpu.sync_copy(src_hbm.at[wid, 0], src_b0)
    pltpu.sync_copy(dst_hbm.at[wid, 0], dst_b0)

    @pl.when(s_id == 0)
    def _():
        pltpu.sync_copy(z_hbm, agg_sh)

    plsc.subcore_barrier()

    for j, (rb, gs, _) in enumerate(ring):
        pltpu.async_copy(h_hbm.at[src_b0.at[j]], rb, gs)

    for b in range(NBLKA):
        sv = src_bufs[b % 2]
        dv = dst_bufs[b % 2]
        svn = src_bufs[(b + 1) % 2]
        dvn = dst_bufs[(b + 1) % 2]
        if b + 1 < NBLKA:
            pltpu.async_copy(src_hbm.at[wid, b + 1], svn, isem)
            pltpu.async_copy(dst_hbm.at[wid, b + 1], dvn, isem)

        def pair(t, carry):
            base = 2 * t
            for j, (rb, gs, ss) in enumerate(ring):
                pltpu.make_async_copy(h_hbm.at[sv.at[base + j]], rb, gs).wait()
                pltpu.async_copy(rb, agg_sh.at[dv.at[base + j]], ss, add=True)

            @pl.when(t + 1 < PAIRA)
            def _():
                for j, (rb, gs, ss) in enumerate(ring):
                    pltpu.make_async_copy(rb, agg_sh.at[dv.at[base + j]], ss).wait()
                    pltpu.async_copy(h_hbm.at[sv.at[base + 2 + j]], rb, gs)

            return carry

        lax.fori_loop(0, PAIRA, pair, 0)

        if b + 1 < NBLKA:
            pltpu.make_async_copy(src_hbm.at[wid, b + 1], svn, isem).wait()
            pltpu.make_async_copy(dst_hbm.at[wid, b + 1], dvn, isem).wait()
            for j, (rb, gs, ss) in enumerate(ring):
                pltpu.make_async_copy(rb, agg_sh.at[dv.at[BCA - 2 + j]], ss).wait()
                pltpu.async_copy(h_hbm.at[svn.at[j]], rb, gs)
        else:
            for j, (rb, gs, ss) in enumerate(ring):
                pltpu.make_async_copy(rb, agg_sh.at[dv.at[BCA - 2 + j]], ss).wait()

    plsc.subcore_barrier()

    pltpu.sync_copy(
        agg_sh.at[pl.ds(s_id * RPT, RPT)],
        out_hbm.at[c_id, pl.ds(s_id * RPT, RPT)],
    )


BN = 1000
NB = N // BN


def _dinv_of(deg_ref):
    deg = deg_ref[0][:, 0:1] + deg_ref[1][:, 0:1] + 1.0
    return lax.rsqrt(deg)


def _tc1_body(deg_ref, x_ref, w_ref, h_ref, hp_ref):
    dinv = _dinv_of(deg_ref)
    h = jnp.dot(x_ref[...], w_ref[...], preferred_element_type=jnp.float32)
    h_ref[...] = h
    hp_ref[...] = h * dinv


def _tc2_body(p_ref, h1_ref, deg_ref, b_ref, w_ref, h2_ref, h2p_ref):
    dinv = _dinv_of(deg_ref)
    a = p_ref[0] + p_ref[1]
    y = a * dinv + h1_ref[...] * (dinv * dinv) + b_ref[...]
    y = jnp.maximum(y, 0.0)
    h2 = jnp.dot(y, w_ref[...], preferred_element_type=jnp.float32)
    h2_ref[...] = h2
    h2p_ref[...] = h2 * dinv


def _tc3_body(p_ref, h2_ref, deg_ref, b_ref, wo_ref, bo_ref, out_ref):
    dinv = _dinv_of(deg_ref)
    a = p_ref[0] + p_ref[1]
    y = a * dinv + h2_ref[...] * (dinv * dinv) + b_ref[...]
    out_ref[...] = (
        jnp.dot(y, wo_ref[...], preferred_element_type=jnp.float32)
        + bo_ref[...]
    )


_deg_spec = pl.BlockSpec((2, BN, DEG_W), lambda i: (0, i, 0))
_row_spec = pl.BlockSpec((BN, D), lambda i: (i, 0))
_p_spec = pl.BlockSpec((2, BN, D), lambda i: (0, i, 0))
_w_spec = pl.BlockSpec((D, D), lambda i: (0, 0))
_b_spec = pl.BlockSpec((1, D), lambda i: (0, 0))

_tc1 = pl.pallas_call(
    _tc1_body,
    grid=(NB,),
    in_specs=[_deg_spec, _row_spec, _w_spec],
    out_specs=[_row_spec, _row_spec],
    out_shape=[
        jax.ShapeDtypeStruct((N, D), jnp.float32),
        jax.ShapeDtypeStruct((N, D), jnp.float32),
    ],
)

_tc2 = pl.pallas_call(
    _tc2_body,
    grid=(NB,),
    in_specs=[_p_spec, _row_spec, _deg_spec, _b_spec, _w_spec],
    out_specs=[_row_spec, _row_spec],
    out_shape=[
        jax.ShapeDtypeStruct((N, D), jnp.float32),
        jax.ShapeDtypeStruct((N, D), jnp.float32),
    ],
)

_tc3 = pl.pallas_call(
    _tc3_body,
    grid=(NB,),
    in_specs=[
        _p_spec, _row_spec, _deg_spec, _b_spec,
        pl.BlockSpec((D, 1), lambda i: (0, 0)),
        pl.BlockSpec((1, 1), lambda i: (0, 0)),
    ],
    out_specs=pl.BlockSpec((BN, 1), lambda i: (i, 0)),
    out_shape=jax.ShapeDtypeStruct((N, 1), jnp.float32),
)


def kernel(x, edge_index, W1, b1, W2, b2, W_out, b_out):
    s2 = edge_index[0].reshape(NW, EPT)
    d2 = edge_index[1].reshape(NW, EPT)
    pad_s = jnp.zeros((NW, EPTP - EPT), jnp.int32)
    pad_d = jnp.full((NW, EPTP - EPT), N, jnp.int32)
    src = jnp.concatenate([s2, pad_s], axis=1).reshape(NW, NBLKA, BCA, KA)
    dst = jnp.concatenate([d2, pad_d], axis=1).reshape(NW, NBLKA, BCA, KA)
    dst_flat = edge_index[1].reshape(NW, NCH, K)
    z128 = jnp.zeros((NPAD, D), jnp.float32)
    z16 = jnp.zeros((NPAD, DEG_W), jnp.float32)

    degp = _deg_kernel(dst_flat, z16)
    h1, h1p = _tc1(degp, x, W1)
    p1 = _agg_kernel(h1p, src, dst, z128)
    h2, h2p = _tc2(p1, h1, degp, b1.reshape(1, D), W2)
    p2 = _agg_kernel(h2p, src, dst, z128)
    out = _tc3(p2, h2, degp, b2.reshape(1, D), W_out, b_out.reshape(1, 1))
    return out


# speedup vs baseline: 1.0004x; 1.0004x over previous
import functools

import jax
import jax.numpy as jnp
from jax import lax
from jax.experimental import pallas as pl
from jax.experimental.pallas import tpu as pltpu
from jax.experimental.pallas import tpu_sc as plsc

N = 10000
E = 320000
D = 128

NC = 2
NS = 16
NW = NC * NS
EPT = E // NW
K = 100
NCH = EPT // K
NPAIR = NCH // 2
NPAD = 10240
RPT = NPAD // NS
DEG_W = 16

_mesh = plsc.VectorSubcoreMesh(core_axis_name="c", subcore_axis_name="s")


@functools.partial(
    pl.kernel,
    out_type=jax.ShapeDtypeStruct((NC, NPAD, DEG_W), jnp.float32),
    mesh=_mesh,
    scratch_types=[
        pltpu.VMEM((NCH, K), jnp.int32),
        pltpu.VMEM((K, DEG_W), jnp.float32),
        pltpu.SemaphoreType.DMA,
        pltpu.VMEM_SHARED((NPAD, DEG_W), jnp.float32),
    ],
)
def _deg_kernel(dst_hbm, z_hbm, out_hbm, dst_v, ones_v, sem, deg_sh):
    c_id = lax.axis_index("c")
    s_id = lax.axis_index("s")
    wid = c_id * NS + s_id

    pltpu.sync_copy(dst_hbm.at[wid], dst_v)

    @pl.when(s_id == 0)
    def _():
        pltpu.sync_copy(z_hbm, deg_sh)

    def fill(i, carry):
        ones_v[i] = jnp.ones((DEG_W,), jnp.float32)
        return carry

    lax.fori_loop(0, K, fill, 0)
    plsc.subcore_barrier()

    def step(i, carry):
        pltpu.async_copy(ones_v, deg_sh.at[dst_v.at[i]], sem, add=True).wait()
        return carry

    lax.fori_loop(0, NCH, step, 0)
    plsc.subcore_barrier()

    pltpu.sync_copy(
        deg_sh.at[pl.ds(s_id * RPT, RPT)],
        out_hbm.at[c_id, pl.ds(s_id * RPT, RPT)],
    )


KA = 100
EPTP = 10400
NCHA = EPTP // KA
NBLKA = 4
BCA = NCHA // NBLKA
PAIRA = BCA // 2


@functools.partial(
    pl.kernel,
    out_type=jax.ShapeDtypeStruct((NC, NPAD, D), jnp.float32),
    mesh=_mesh,
    scratch_types=[
        pltpu.VMEM((BCA, KA), jnp.int32),
        pltpu.VMEM((BCA, KA), jnp.int32),
        pltpu.VMEM((BCA, KA), jnp.int32),
        pltpu.VMEM((BCA, KA), jnp.int32),
        pltpu.VMEM((KA, D), jnp.float32),
        pltpu.VMEM((KA, D), jnp.float32),
        pltpu.SemaphoreType.DMA,
        pltpu.SemaphoreType.DMA,
        pltpu.SemaphoreType.DMA,
        pltpu.SemaphoreType.DMA,
        pltpu.SemaphoreType.DMA,
        pltpu.VMEM_SHARED((NPAD, D), jnp.float32),
    ],
)
def _agg_kernel(h_hbm, src_hbm, dst_hbm, z_hbm, out_hbm,
                src_b0, src_b1, dst_b0, dst_b1, rows0, rows1,
                gsem0, gsem1, ssem0, ssem1, isem, agg_sh):
    src_bufs = (src_b0, src_b1)
    dst_bufs = (dst_b0, dst_b1)
    ring = ((rows0, gsem0, ssem0), (rows1, gsem1, ssem1))
    c_id = lax.axis_index("c")
    s_id = lax.axis_index("s")
    wid = c_id * NS + s_id

    pltpu.sync_copy(src_hbm.at[wid, 0], src_b0)
    pltpu.sync_copy(dst_hbm.at[wid, 0], dst_b0)

    @pl.when(s_id == 0)
    def _():
        pltpu.sync_copy(z_hbm, agg_sh)

    plsc.subcore_barrier()

    for j, (rb, gs, _) in enumerate(ring):
        pltpu.async_copy(h_hbm.at[src_b0.at[j]], rb, gs)

    for b in range(NBLKA):
        sv = src_bufs[b % 2]
        dv = dst_bufs[b % 2]
        svn = src_bufs[(b + 1) % 2]
        dvn = dst_bufs[(b + 1) % 2]
        if b + 1 < NBLKA:
            pltpu.async_copy(src_hbm.at[wid, b + 1], svn, isem)
            pltpu.async_copy(dst_hbm.at[wid, b + 1], dvn, isem)

        def pair(t, carry):
            base = 2 * t
            for j, (rb, gs, ss) in enumerate(ring):
                pltpu.make_async_copy(h_hbm.at[sv.at[base + j]], rb, gs).wait()
                pltpu.async_copy(rb, agg_sh.at[dv.at[base + j]], ss, add=True)

            @pl.when(t + 1 < PAIRA)
            def _():
                for j, (rb, gs, ss) in enumerate(ring):
                    pltpu.make_async_copy(rb, agg_sh.at[dv.at[base + j]], ss).wait()
                    pltpu.async_copy(h_hbm.at[sv.at[base + 2 + j]], rb, gs)

            return carry

        lax.fori_loop(0, PAIRA, pair, 0)

        if b + 1 < NBLKA:
            pltpu.make_async_copy(src_hbm.at[wid, b + 1], svn, isem).wait()
            pltpu.make_async_copy(dst_hbm.at[wid, b + 1], dvn, isem).wait()
            for j, (rb, gs, ss) in enumerate(ring):
                pltpu.make_async_copy(rb, agg_sh.at[dv.at[BCA - 2 + j]], ss).wait()
                pltpu.async_copy(h_hbm.at[svn.at[j]], rb, gs)
        else:
            for j, (rb, gs, ss) in enumerate(ring):
                pltpu.make_async_copy(rb, agg_sh.at[dv.at[BCA - 2 + j]], ss).wait()

    plsc.subcore_barrier()

    pltpu.sync_copy(
        agg_sh.at[pl.ds(s_id * RPT, RPT)],
        out_hbm.at[c_id, pl.ds(s_id * RPT, RPT)],
    )


BN = 1000
NB = N // BN


def _dinv_of(deg_ref):
    deg = deg_ref[0][:, 0:1] + deg_ref[1][:, 0:1] + 1.0
    return lax.rsqrt(deg)


def _tc1_body(deg_ref, x_ref, w_ref, h_ref, hp_ref):
    dinv = _dinv_of(deg_ref)
    h = jnp.dot(x_ref[...], w_ref[...], preferred_element_type=jnp.float32)
    h_ref[...] = h
    hp_ref[...] = h * dinv


def _tc2_body(p_ref, h1_ref, deg_ref, b_ref, w_ref, h2_ref, h2p_ref):
    dinv = _dinv_of(deg_ref)
    a = p_ref[0] + p_ref[1]
    y = a * dinv + h1_ref[...] * (dinv * dinv) + b_ref[...]
    y = jnp.maximum(y, 0.0)
    h2 = jnp.dot(y, w_ref[...], preferred_element_type=jnp.float32)
    h2_ref[...] = h2
    h2p_ref[...] = h2 * dinv


def _tc3_body(p_ref, h2_ref, deg_ref, b_ref, wo_ref, bo_ref, out_ref):
    dinv = _dinv_of(deg_ref)
    a = p_ref[0] + p_ref[1]
    y = a * dinv + h2_ref[...] * (dinv * dinv) + b_ref[...]
    out_ref[...] = (
        jnp.dot(y, wo_ref[...], preferred_element_type=jnp.float32)
        + bo_ref[...]
    )


_deg_spec = pl.BlockSpec((2, BN, DEG_W), lambda i: (0, i, 0))
_row_spec = pl.BlockSpec((BN, D), lambda i: (i, 0))
_p_spec = pl.BlockSpec((2, BN, D), lambda i: (0, i, 0))
_w_spec = pl.BlockSpec((D, D), lambda i: (0, 0))
_b_spec = pl.BlockSpec((1, D), lambda i: (0, 0))

_tc1 = pl.pallas_call(
    _tc1_body,
    grid=(NB,),
    in_specs=[_deg_spec, _row_spec, _w_spec],
    out_specs=[_row_spec, _row_spec],
    out_shape=[
        jax.ShapeDtypeStruct((N, D), jnp.float32),
        jax.ShapeDtypeStruct((N, D), jnp.float32),
    ],
)

_tc2 = pl.pallas_call(
    _tc2_body,
    grid=(NB,),
    in_specs=[_p_spec, _row_spec, _deg_spec, _b_spec, _w_spec],
    out_specs=[_row_spec, _row_spec],
    out_shape=[
        jax.ShapeDtypeStruct((N, D), jnp.float32),
        jax.ShapeDtypeStruct((N, D), jnp.float32),
    ],
)

_tc3 = pl.pallas_call(
    _tc3_body,
    grid=(NB,),
    in_specs=[
        _p_spec, _row_spec, _deg_spec, _b_spec,
        pl.BlockSpec((D, 1), lambda i: (0, 0)),
        pl.BlockSpec((1, 1), lambda i: (0, 0)),
    ],
    out_specs=pl.BlockSpec((BN, 1), lambda i: (i, 0)),
    out_shape=jax.ShapeDtypeStruct((N, 1), jnp.float32),
)


def kernel(x, edge_index, W1, b1, W2, b2, W_out, b_out):
    s2 = edge_index[0].reshape(NW, EPT)
    d2 = edge_index[1].reshape(NW, EPT)
    pad_s = jnp.zeros((NW, EPTP - EPT), jnp.int32)
    pad_d = N + jnp.arange(NW * (EPTP - EPT), dtype=jnp.int32).reshape(
        NW, EPTP - EPT) % (NPAD - N)
    src = jnp.concatenate([s2, pad_s], axis=1).reshape(NW, NBLKA, BCA, KA)
    dst = jnp.concatenate([d2, pad_d], axis=1).reshape(NW, NBLKA, BCA, KA)
    dst_flat = edge_index[1].reshape(NW, NCH, K)
    z128 = jnp.zeros((NPAD, D), jnp.float32)
    z16 = jnp.zeros((NPAD, DEG_W), jnp.float32)

    degp = _deg_kernel(dst_flat, z16)
    h1, h1p = _tc1(degp, x, W1)
    p1 = _agg_kernel(h1p, src, dst, z128)
    h2, h2p = _tc2(p1, h1, degp, b1.reshape(1, D), W2)
    p2 = _agg_kernel(h2p, src, dst, z128)
    out = _tc3(p2, h2, degp, b2.reshape(1, D), W_out, b_out.reshape(1, 1))
    return out


# speedup vs baseline: 3.3795x; 3.3783x over previous
import functools

import jax
import jax.numpy as jnp
from jax import lax
from jax.experimental import pallas as pl
from jax.experimental.pallas import tpu as pltpu
from jax.experimental.pallas import tpu_sc as plsc

N = 10000
E = 320000
D = 128

NC = 2
NS = 16
NW = NC * NS
EPT = E // NW
K = 100
NCH = EPT // K
NPAIR = NCH // 2
NPAD = 10240
RPT = NPAD // NS
DEG_W = 16

_mesh = plsc.VectorSubcoreMesh(core_axis_name="c", subcore_axis_name="s")


@functools.partial(
    pl.kernel,
    out_type=jax.ShapeDtypeStruct((NC, NPAD, DEG_W), jnp.float32),
    mesh=_mesh,
    scratch_types=[
        pltpu.VMEM((NCH, K), jnp.int32),
        pltpu.VMEM((K, DEG_W), jnp.float32),
        pltpu.SemaphoreType.DMA,
        pltpu.VMEM_SHARED((NPAD, DEG_W), jnp.float32),
    ],
)
def _deg_kernel(dst_hbm, z_hbm, out_hbm, dst_v, ones_v, sem, deg_sh):
    c_id = lax.axis_index("c")
    s_id = lax.axis_index("s")
    wid = c_id * NS + s_id

    pltpu.sync_copy(dst_hbm.at[wid], dst_v)

    @pl.when(s_id == 0)
    def _():
        pltpu.sync_copy(z_hbm, deg_sh)

    def fill(i, carry):
        ones_v[i] = jnp.ones((DEG_W,), jnp.float32)
        return carry

    lax.fori_loop(0, K, fill, 0)
    plsc.subcore_barrier()

    def step(i, carry):
        pltpu.async_copy(ones_v, deg_sh.at[dst_v.at[i]], sem, add=True).wait()
        return carry

    lax.fori_loop(0, NCH, step, 0)
    plsc.subcore_barrier()

    pltpu.sync_copy(
        deg_sh.at[pl.ds(s_id * RPT, RPT)],
        out_hbm.at[c_id, pl.ds(s_id * RPT, RPT)],
    )


KA = 125
NCHA = EPT // KA
NBLKA = 4
BCA = NCHA // NBLKA
PAIRA = BCA // 2


@functools.partial(
    pl.kernel,
    out_type=jax.ShapeDtypeStruct((NC, NPAD, D), jnp.float32),
    mesh=_mesh,
    scratch_types=[
        pltpu.VMEM((BCA, KA), jnp.int32),
        pltpu.VMEM((BCA, KA), jnp.int32),
        pltpu.VMEM((BCA, KA), jnp.int32),
        pltpu.VMEM((BCA, KA), jnp.int32),
        pltpu.VMEM((KA, D), jnp.float32),
        pltpu.VMEM((KA, D), jnp.float32),
        pltpu.SemaphoreType.DMA,
        pltpu.SemaphoreType.DMA,
        pltpu.SemaphoreType.DMA,
        pltpu.SemaphoreType.DMA,
        pltpu.SemaphoreType.DMA,
        pltpu.VMEM_SHARED((NPAD, D), jnp.float32),
    ],
)
def _agg_kernel(h_hbm, src_hbm, dst_hbm, z_hbm, out_hbm,
                src_b0, src_b1, dst_b0, dst_b1, rows0, rows1,
                gsem0, gsem1, ssem0, ssem1, isem, agg_sh):
    src_bufs = (src_b0, src_b1)
    dst_bufs = (dst_b0, dst_b1)
    ring = ((rows0, gsem0, ssem0), (rows1, gsem1, ssem1))
    c_id = lax.axis_index("c")
    s_id = lax.axis_index("s")
    wid = c_id * NS + s_id

    pltpu.sync_copy(src_hbm.at[wid, 0], src_b0)
    pltpu.sync_copy(dst_hbm.at[wid, 0], dst_b0)

    @pl.when(s_id == 0)
    def _():
        pltpu.sync_copy(z_hbm, agg_sh)

    plsc.subcore_barrier()

    for j, (rb, gs, _) in enumerate(ring):
        pltpu.async_copy(h_hbm.at[src_b0.at[j]], rb, gs)

    for b in range(NBLKA):
        sv = src_bufs[b % 2]
        dv = dst_bufs[b % 2]
        svn = src_bufs[(b + 1) % 2]
        dvn = dst_bufs[(b + 1) % 2]
        if b + 1 < NBLKA:
            pltpu.async_copy(src_hbm.at[wid, b + 1], svn, isem)
            pltpu.async_copy(dst_hbm.at[wid, b + 1], dvn, isem)

        def pair(t, carry):
            base = 2 * t
            for j, (rb, gs, ss) in enumerate(ring):
                pltpu.make_async_copy(h_hbm.at[sv.at[base + j]], rb, gs).wait()
                pltpu.async_copy(rb, agg_sh.at[dv.at[base + j]], ss, add=True)

            @pl.when(t + 1 < PAIRA)
            def _():
                for j, (rb, gs, ss) in enumerate(ring):
                    pltpu.make_async_copy(rb, agg_sh.at[dv.at[base + j]], ss).wait()
                    pltpu.async_copy(h_hbm.at[sv.at[base + 2 + j]], rb, gs)

            return carry

        lax.fori_loop(0, PAIRA, pair, 0)

        if b + 1 < NBLKA:
            pltpu.make_async_copy(src_hbm.at[wid, b + 1], svn, isem).wait()
            pltpu.make_async_copy(dst_hbm.at[wid, b + 1], dvn, isem).wait()
            for j, (rb, gs, ss) in enumerate(ring):
                pltpu.make_async_copy(rb, agg_sh.at[dv.at[BCA - 2 + j]], ss).wait()
                pltpu.async_copy(h_hbm.at[svn.at[j]], rb, gs)
        else:
            for j, (rb, gs, ss) in enumerate(ring):
                pltpu.make_async_copy(rb, agg_sh.at[dv.at[BCA - 2 + j]], ss).wait()

    plsc.subcore_barrier()

    pltpu.sync_copy(
        agg_sh.at[pl.ds(s_id * RPT, RPT)],
        out_hbm.at[c_id, pl.ds(s_id * RPT, RPT)],
    )


BN = 1000
NB = N // BN


def _dinv_of(deg_ref):
    deg = deg_ref[0][:, 0:1] + deg_ref[1][:, 0:1] + 1.0
    return lax.rsqrt(deg)


def _tc1_body(deg_ref, x_ref, w_ref, h_ref, hp_ref):
    dinv = _dinv_of(deg_ref)
    h = jnp.dot(x_ref[...], w_ref[...], preferred_element_type=jnp.float32)
    h_ref[...] = h
    hp_ref[...] = h * dinv


def _tc2_body(p_ref, h1_ref, deg_ref, b_ref, w_ref, h2_ref, h2p_ref):
    dinv = _dinv_of(deg_ref)
    a = p_ref[0] + p_ref[1]
    y = a * dinv + h1_ref[...] * (dinv * dinv) + b_ref[...]
    y = jnp.maximum(y, 0.0)
    h2 = jnp.dot(y, w_ref[...], preferred_element_type=jnp.float32)
    h2_ref[...] = h2
    h2p_ref[...] = h2 * dinv


def _tc3_body(p_ref, h2_ref, deg_ref, b_ref, wo_ref, bo_ref, out_ref):
    dinv = _dinv_of(deg_ref)
    a = p_ref[0] + p_ref[1]
    y = a * dinv + h2_ref[...] * (dinv * dinv) + b_ref[...]
    out_ref[...] = (
        jnp.dot(y, wo_ref[...], preferred_element_type=jnp.float32)
        + bo_ref[...]
    )


_deg_spec = pl.BlockSpec((2, BN, DEG_W), lambda i: (0, i, 0))
_row_spec = pl.BlockSpec((BN, D), lambda i: (i, 0))
_p_spec = pl.BlockSpec((2, BN, D), lambda i: (0, i, 0))
_w_spec = pl.BlockSpec((D, D), lambda i: (0, 0))
_b_spec = pl.BlockSpec((1, D), lambda i: (0, 0))

_tc1 = pl.pallas_call(
    _tc1_body,
    grid=(NB,),
    in_specs=[_deg_spec, _row_spec, _w_spec],
    out_specs=[_row_spec, _row_spec],
    out_shape=[
        jax.ShapeDtypeStruct((N, D), jnp.float32),
        jax.ShapeDtypeStruct((N, D), jnp.float32),
    ],
)

_tc2 = pl.pallas_call(
    _tc2_body,
    grid=(NB,),
    in_specs=[_p_spec, _row_spec, _deg_spec, _b_spec, _w_spec],
    out_specs=[_row_spec, _row_spec],
    out_shape=[
        jax.ShapeDtypeStruct((N, D), jnp.float32),
        jax.ShapeDtypeStruct((N, D), jnp.float32),
    ],
)

_tc3 = pl.pallas_call(
    _tc3_body,
    grid=(NB,),
    in_specs=[
        _p_spec, _row_spec, _deg_spec, _b_spec,
        pl.BlockSpec((D, 1), lambda i: (0, 0)),
        pl.BlockSpec((1, 1), lambda i: (0, 0)),
    ],
    out_specs=pl.BlockSpec((BN, 1), lambda i: (i, 0)),
    out_shape=jax.ShapeDtypeStruct((N, 1), jnp.float32),
)


def kernel(x, edge_index, W1, b1, W2, b2, W_out, b_out):
    src = edge_index[0].reshape(NW, NBLKA, BCA, KA)
    dst = edge_index[1].reshape(NW, NBLKA, BCA, KA)
    dst_flat = edge_index[1].reshape(NW, NCH, K)
    z128 = jnp.zeros((NPAD, D), jnp.float32)
    z16 = jnp.zeros((NPAD, DEG_W), jnp.float32)

    degp = _deg_kernel(dst_flat, z16)
    h1, h1p = _tc1(degp, x, W1)
    p1 = _agg_kernel(h1p, src, dst, z128)
    h2, h2p = _tc2(p1, h1, degp, b1.reshape(1, D), W2)
    p2 = _agg_kernel(h2p, src, dst, z128)
    out = _tc3(p2, h2, degp, b2.reshape(1, D), W_out, b_out.reshape(1, 1))
    return out


# speedup vs baseline: 3.4385x; 1.0175x over previous
import functools

import jax
import jax.numpy as jnp
from jax import lax
from jax.experimental import pallas as pl
from jax.experimental.pallas import tpu as pltpu
from jax.experimental.pallas import tpu_sc as plsc

N = 10000
E = 320000
D = 128

NC = 2
NS = 16
NW = NC * NS
EPT = E // NW
K = 100
NCH = EPT // K
NPAIR = NCH // 2
NPAD = 10240
RPT = NPAD // NS
DEG_W = 16

_mesh = plsc.VectorSubcoreMesh(core_axis_name="c", subcore_axis_name="s")


KD = 125
NCHD = EPT // KD
DEG_WIN = 16


@functools.partial(
    pl.kernel,
    out_type=jax.ShapeDtypeStruct((NC, NPAD, DEG_W), jnp.float32),
    mesh=_mesh,
    scratch_types=[
        pltpu.VMEM((NCHD, KD), jnp.int32),
        pltpu.VMEM((KD, DEG_W), jnp.float32),
        pltpu.SemaphoreType.DMA,
        pltpu.VMEM_SHARED((NPAD, DEG_W), jnp.float32),
    ],
)
def _deg_kernel(dst_hbm, z_hbm, out_hbm, dst_v, ones_v, sem, deg_sh):
    c_id = lax.axis_index("c")
    s_id = lax.axis_index("s")
    wid = c_id * NS + s_id

    pltpu.sync_copy(dst_hbm.at[wid], dst_v)

    @pl.when(s_id == 0)
    def _():
        pltpu.sync_copy(z_hbm, deg_sh)

    def fill(i, carry):
        ones_v[i] = jnp.ones((DEG_W,), jnp.float32)
        return carry

    lax.fori_loop(0, KD, fill, 0)
    plsc.subcore_barrier()

    def step(i, carry):
        pltpu.async_copy(ones_v, deg_sh.at[dst_v.at[i]], sem, add=True)

        @pl.when(i >= DEG_WIN)
        def _():
            pltpu.make_async_copy(ones_v, deg_sh.at[dst_v.at[0]], sem).wait()

        return carry

    lax.fori_loop(0, NCHD, step, 0)

    def drain(i, carry):
        pltpu.make_async_copy(ones_v, deg_sh.at[dst_v.at[0]], sem).wait()
        return carry

    lax.fori_loop(0, DEG_WIN, drain, 0)
    plsc.subcore_barrier()

    pltpu.sync_copy(
        deg_sh.at[pl.ds(s_id * RPT, RPT)],
        out_hbm.at[c_id, pl.ds(s_id * RPT, RPT)],
    )


KA = 125
NCHA = EPT // KA
NBLKA = 4
BCA = NCHA // NBLKA
PAIRA = BCA // 2


@functools.partial(
    pl.kernel,
    out_type=jax.ShapeDtypeStruct((NC, NPAD, D), jnp.float32),
    mesh=_mesh,
    scratch_types=[
        pltpu.VMEM((BCA, KA), jnp.int32),
        pltpu.VMEM((BCA, KA), jnp.int32),
        pltpu.VMEM((BCA, KA), jnp.int32),
        pltpu.VMEM((BCA, KA), jnp.int32),
        pltpu.VMEM((KA, D), jnp.float32),
        pltpu.VMEM((KA, D), jnp.float32),
        pltpu.SemaphoreType.DMA,
        pltpu.SemaphoreType.DMA,
        pltpu.SemaphoreType.DMA,
        pltpu.SemaphoreType.DMA,
        pltpu.SemaphoreType.DMA,
        pltpu.VMEM_SHARED((NPAD, D), jnp.float32),
    ],
)
def _agg_kernel(h_hbm, src_hbm, dst_hbm, z_hbm, out_hbm,
                src_b0, src_b1, dst_b0, dst_b1, rows0, rows1,
                gsem0, gsem1, ssem0, ssem1, isem, agg_sh):
    src_bufs = (src_b0, src_b1)
    dst_bufs = (dst_b0, dst_b1)
    ring = ((rows0, gsem0, ssem0), (rows1, gsem1, ssem1))
    c_id = lax.axis_index("c")
    s_id = lax.axis_index("s")
    wid = c_id * NS + s_id

    pltpu.sync_copy(src_hbm.at[wid, 0], src_b0)
    pltpu.sync_copy(dst_hbm.at[wid, 0], dst_b0)

    @pl.when(s_id == 0)
    def _():
        pltpu.sync_copy(z_hbm, agg_sh)

    plsc.subcore_barrier()

    for j, (rb, gs, _) in enumerate(ring):
        pltpu.async_copy(h_hbm.at[src_b0.at[j]], rb, gs)

    for b in range(NBLKA):
        sv = src_bufs[b % 2]
        dv = dst_bufs[b % 2]
        svn = src_bufs[(b + 1) % 2]
        dvn = dst_bufs[(b + 1) % 2]
        if b + 1 < NBLKA:
            pltpu.async_copy(src_hbm.at[wid, b + 1], svn, isem)
            pltpu.async_copy(dst_hbm.at[wid, b + 1], dvn, isem)

        def pair(t, carry):
            base = 2 * t
            for j, (rb, gs, ss) in enumerate(ring):
                pltpu.make_async_copy(h_hbm.at[sv.at[base + j]], rb, gs).wait()
                pltpu.async_copy(rb, agg_sh.at[dv.at[base + j]], ss, add=True)

            @pl.when(t + 1 < PAIRA)
            def _():
                for j, (rb, gs, ss) in enumerate(ring):
                    pltpu.make_async_copy(rb, agg_sh.at[dv.at[base + j]], ss).wait()
                    pltpu.async_copy(h_hbm.at[sv.at[base + 2 + j]], rb, gs)

            return carry

        lax.fori_loop(0, PAIRA, pair, 0)

        if b + 1 < NBLKA:
            pltpu.make_async_copy(src_hbm.at[wid, b + 1], svn, isem).wait()
            pltpu.make_async_copy(dst_hbm.at[wid, b + 1], dvn, isem).wait()
            for j, (rb, gs, ss) in enumerate(ring):
                pltpu.make_async_copy(rb, agg_sh.at[dv.at[BCA - 2 + j]], ss).wait()
                pltpu.async_copy(h_hbm.at[svn.at[j]], rb, gs)
        else:
            for j, (rb, gs, ss) in enumerate(ring):
                pltpu.make_async_copy(rb, agg_sh.at[dv.at[BCA - 2 + j]], ss).wait()

    plsc.subcore_barrier()

    pltpu.sync_copy(
        agg_sh.at[pl.ds(s_id * RPT, RPT)],
        out_hbm.at[c_id, pl.ds(s_id * RPT, RPT)],
    )


BN = 1000
NB = N // BN


def _dinv_of(deg_ref):
    deg = deg_ref[0][:, 0:1] + deg_ref[1][:, 0:1] + 1.0
    return lax.rsqrt(deg)


def _tc1a_body(x_ref, w_ref, h_ref):
    h_ref[...] = jnp.dot(
        x_ref[...], w_ref[...], preferred_element_type=jnp.float32)


def _tc1b_body(deg_ref, h_ref, hp_ref):
    hp_ref[...] = h_ref[...] * _dinv_of(deg_ref)


def _tc2_body(p_ref, h1_ref, deg_ref, b_ref, w_ref, h2_ref, h2p_ref):
    dinv = _dinv_of(deg_ref)
    a = p_ref[0] + p_ref[1]
    y = a * dinv + h1_ref[...] * (dinv * dinv) + b_ref[...]
    y = jnp.maximum(y, 0.0)
    h2 = jnp.dot(y, w_ref[...], preferred_element_type=jnp.float32)
    h2_ref[...] = h2
    h2p_ref[...] = h2 * dinv


def _tc3_body(p_ref, h2_ref, deg_ref, b_ref, wo_ref, bo_ref, out_ref):
    dinv = _dinv_of(deg_ref)
    a = p_ref[0] + p_ref[1]
    y = a * dinv + h2_ref[...] * (dinv * dinv) + b_ref[...]
    out_ref[...] = (
        jnp.dot(y, wo_ref[...], preferred_element_type=jnp.float32)
        + bo_ref[...]
    )


_deg_spec = pl.BlockSpec((2, BN, DEG_W), lambda i: (0, i, 0))
_row_spec = pl.BlockSpec((BN, D), lambda i: (i, 0))
_p_spec = pl.BlockSpec((2, BN, D), lambda i: (0, i, 0))
_w_spec = pl.BlockSpec((D, D), lambda i: (0, 0))
_b_spec = pl.BlockSpec((1, D), lambda i: (0, 0))

_tc1a = pl.pallas_call(
    _tc1a_body,
    grid=(NB,),
    in_specs=[_row_spec, _w_spec],
    out_specs=_row_spec,
    out_shape=jax.ShapeDtypeStruct((N, D), jnp.float32),
)

_tc1b = pl.pallas_call(
    _tc1b_body,
    grid=(NB,),
    in_specs=[_deg_spec, _row_spec],
    out_specs=_row_spec,
    out_shape=jax.ShapeDtypeStruct((N, D), jnp.float32),
)

_tc2 = pl.pallas_call(
    _tc2_body,
    grid=(NB,),
    in_specs=[_p_spec, _row_spec, _deg_spec, _b_spec, _w_spec],
    out_specs=[_row_spec, _row_spec],
    out_shape=[
        jax.ShapeDtypeStruct((N, D), jnp.float32),
        jax.ShapeDtypeStruct((N, D), jnp.float32),
    ],
)

_tc3 = pl.pallas_call(
    _tc3_body,
    grid=(NB,),
    in_specs=[
        _p_spec, _row_spec, _deg_spec, _b_spec,
        pl.BlockSpec((D, 1), lambda i: (0, 0)),
        pl.BlockSpec((1, 1), lambda i: (0, 0)),
    ],
    out_specs=pl.BlockSpec((BN, 1), lambda i: (i, 0)),
    out_shape=jax.ShapeDtypeStruct((N, 1), jnp.float32),
)


def kernel(x, edge_index, W1, b1, W2, b2, W_out, b_out):
    src = edge_index[0].reshape(NW, NBLKA, BCA, KA)
    dst = edge_index[1].reshape(NW, NBLKA, BCA, KA)
    dst_deg = edge_index[1].reshape(NW, NCHD, KD)
    z128 = jnp.zeros((NPAD, D), jnp.float32)
    z16 = jnp.zeros((NPAD, DEG_W), jnp.float32)

    degp = _deg_kernel(dst_deg, z16)
    h1 = _tc1a(x, W1)
    h1p = _tc1b(degp, h1)
    p1 = _agg_kernel(h1p, src, dst, z128)
    h2, h2p = _tc2(p1, h1, degp, b1.reshape(1, D), W2)
    p2 = _agg_kernel(h2p, src, dst, z128)
    out = _tc3(p2, h2, degp, b2.reshape(1, D), W_out, b_out.reshape(1, 1))
    return out


# speedup vs baseline: 3.4389x; 1.0001x over previous
import functools

import jax
import jax.numpy as jnp
from jax import lax
from jax.experimental import pallas as pl
from jax.experimental.pallas import tpu as pltpu
from jax.experimental.pallas import tpu_sc as plsc

N = 10000
E = 320000
D = 128

NC = 2
NS = 16
NW = NC * NS
EPT = E // NW
K = 100
NCH = EPT // K
NPAIR = NCH // 2
NPAD = 10240
RPT = NPAD // NS
DEG_W = 16

_mesh = plsc.VectorSubcoreMesh(core_axis_name="c", subcore_axis_name="s")


KD = 125
NCHD = EPT // KD
DEG_WIN = 2


@functools.partial(
    pl.kernel,
    out_type=jax.ShapeDtypeStruct((NC, NPAD, DEG_W), jnp.float32),
    mesh=_mesh,
    scratch_types=[
        pltpu.VMEM((NCHD, KD), jnp.int32),
        pltpu.VMEM((KD, DEG_W), jnp.float32),
        pltpu.SemaphoreType.DMA,
        pltpu.VMEM_SHARED((NPAD, DEG_W), jnp.float32),
    ],
)
def _deg_kernel(dst_hbm, z_hbm, out_hbm, dst_v, ones_v, sem, deg_sh):
    c_id = lax.axis_index("c")
    s_id = lax.axis_index("s")
    wid = c_id * NS + s_id

    pltpu.sync_copy(dst_hbm.at[wid], dst_v)

    @pl.when(s_id == 0)
    def _():
        pltpu.sync_copy(z_hbm, deg_sh)

    def fill(i, carry):
        ones_v[i] = jnp.ones((DEG_W,), jnp.float32)
        return carry

    lax.fori_loop(0, KD, fill, 0)
    plsc.subcore_barrier()

    def step(i, carry):
        pltpu.async_copy(ones_v, deg_sh.at[dst_v.at[i]], sem, add=True)

        @pl.when(i >= DEG_WIN)
        def _():
            pltpu.make_async_copy(ones_v, deg_sh.at[dst_v.at[0]], sem).wait()

        return carry

    lax.fori_loop(0, NCHD, step, 0)

    def drain(i, carry):
        pltpu.make_async_copy(ones_v, deg_sh.at[dst_v.at[0]], sem).wait()
        return carry

    lax.fori_loop(0, DEG_WIN, drain, 0)
    plsc.subcore_barrier()

    pltpu.sync_copy(
        deg_sh.at[pl.ds(s_id * RPT, RPT)],
        out_hbm.at[c_id, pl.ds(s_id * RPT, RPT)],
    )


KA = 125
NCHA = EPT // KA
NBLKA = 4
BCA = NCHA // NBLKA
PAIRA = BCA // 2


@functools.partial(
    pl.kernel,
    out_type=jax.ShapeDtypeStruct((NC, NPAD, D), jnp.float32),
    mesh=_mesh,
    scratch_types=[
        pltpu.VMEM((BCA, KA), jnp.int32),
        pltpu.VMEM((BCA, KA), jnp.int32),
        pltpu.VMEM((BCA, KA), jnp.int32),
        pltpu.VMEM((BCA, KA), jnp.int32),
        pltpu.VMEM((KA, D), jnp.float32),
        pltpu.VMEM((KA, D), jnp.float32),
        pltpu.SemaphoreType.DMA,
        pltpu.SemaphoreType.DMA,
        pltpu.SemaphoreType.DMA,
        pltpu.SemaphoreType.DMA,
        pltpu.SemaphoreType.DMA,
        pltpu.VMEM_SHARED((NPAD, D), jnp.float32),
    ],
)
def _agg_kernel(h_hbm, src_hbm, dst_hbm, z_hbm, out_hbm,
                src_b0, src_b1, dst_b0, dst_b1, rows0, rows1,
                gsem0, gsem1, ssem0, ssem1, isem, agg_sh):
    src_bufs = (src_b0, src_b1)
    dst_bufs = (dst_b0, dst_b1)
    ring = ((rows0, gsem0, ssem0), (rows1, gsem1, ssem1))
    c_id = lax.axis_index("c")
    s_id = lax.axis_index("s")
    wid = c_id * NS + s_id

    pltpu.sync_copy(src_hbm.at[wid, 0], src_b0)
    pltpu.sync_copy(dst_hbm.at[wid, 0], dst_b0)

    @pl.when(s_id == 0)
    def _():
        pltpu.sync_copy(z_hbm, agg_sh)

    plsc.subcore_barrier()

    for j, (rb, gs, _) in enumerate(ring):
        pltpu.async_copy(h_hbm.at[src_b0.at[j]], rb, gs)

    for b in range(NBLKA):
        sv = src_bufs[b % 2]
        dv = dst_bufs[b % 2]
        svn = src_bufs[(b + 1) % 2]
        dvn = dst_bufs[(b + 1) % 2]
        if b + 1 < NBLKA:
            pltpu.async_copy(src_hbm.at[wid, b + 1], svn, isem)
            pltpu.async_copy(dst_hbm.at[wid, b + 1], dvn, isem)

        def pair(t, carry):
            base = 2 * t
            for j, (rb, gs, ss) in enumerate(ring):
                pltpu.make_async_copy(h_hbm.at[sv.at[base + j]], rb, gs).wait()
                pltpu.async_copy(rb, agg_sh.at[dv.at[base + j]], ss, add=True)

            @pl.when(t + 1 < PAIRA)
            def _():
                for j, (rb, gs, ss) in enumerate(ring):
                    pltpu.make_async_copy(rb, agg_sh.at[dv.at[base + j]], ss).wait()
                    pltpu.async_copy(h_hbm.at[sv.at[base + 2 + j]], rb, gs)

            return carry

        lax.fori_loop(0, PAIRA, pair, 0)

        if b + 1 < NBLKA:
            pltpu.make_async_copy(src_hbm.at[wid, b + 1], svn, isem).wait()
            pltpu.make_async_copy(dst_hbm.at[wid, b + 1], dvn, isem).wait()
            for j, (rb, gs, ss) in enumerate(ring):
                pltpu.make_async_copy(rb, agg_sh.at[dv.at[BCA - 2 + j]], ss).wait()
                pltpu.async_copy(h_hbm.at[svn.at[j]], rb, gs)
        else:
            for j, (rb, gs, ss) in enumerate(ring):
                pltpu.make_async_copy(rb, agg_sh.at[dv.at[BCA - 2 + j]], ss).wait()

    plsc.subcore_barrier()

    pltpu.sync_copy(
        agg_sh.at[pl.ds(s_id * RPT, RPT)],
        out_hbm.at[c_id, pl.ds(s_id * RPT, RPT)],
    )


BN = 1000
NB = N // BN


def _dinv_of(deg_ref):
    deg = deg_ref[0][:, 0:1] + deg_ref[1][:, 0:1] + 1.0
    return lax.rsqrt(deg)


def _tc1a_body(x_ref, w_ref, h_ref):
    h_ref[...] = jnp.dot(
        x_ref[...], w_ref[...], preferred_element_type=jnp.float32)


def _tc1b_body(deg_ref, h_ref, hp_ref):
    hp_ref[...] = h_ref[...] * _dinv_of(deg_ref)


def _tc2_body(p_ref, h1_ref, deg_ref, b_ref, w_ref, h2_ref, h2p_ref):
    dinv = _dinv_of(deg_ref)
    a = p_ref[0] + p_ref[1]
    y = a * dinv + h1_ref[...] * (dinv * dinv) + b_ref[...]
    y = jnp.maximum(y, 0.0)
    h2 = jnp.dot(y, w_ref[...], preferred_element_type=jnp.float32)
    h2_ref[...] = h2
    h2p_ref[...] = h2 * dinv


def _tc3_body(p_ref, h2_ref, deg_ref, b_ref, wo_ref, bo_ref, out_ref):
    dinv = _dinv_of(deg_ref)
    a = p_ref[0] + p_ref[1]
    y = a * dinv + h2_ref[...] * (dinv * dinv) + b_ref[...]
    out_ref[...] = (
        jnp.dot(y, wo_ref[...], preferred_element_type=jnp.float32)
        + bo_ref[...]
    )


_deg_spec = pl.BlockSpec((2, BN, DEG_W), lambda i: (0, i, 0))
_row_spec = pl.BlockSpec((BN, D), lambda i: (i, 0))
_p_spec = pl.BlockSpec((2, BN, D), lambda i: (0, i, 0))
_w_spec = pl.BlockSpec((D, D), lambda i: (0, 0))
_b_spec = pl.BlockSpec((1, D), lambda i: (0, 0))

_tc1a = pl.pallas_call(
    _tc1a_body,
    grid=(NB,),
    in_specs=[_row_spec, _w_spec],
    out_specs=_row_spec,
    out_shape=jax.ShapeDtypeStruct((N, D), jnp.float32),
)

_tc1b = pl.pallas_call(
    _tc1b_body,
    grid=(NB,),
    in_specs=[_deg_spec, _row_spec],
    out_specs=_row_spec,
    out_shape=jax.ShapeDtypeStruct((N, D), jnp.float32),
)

_tc2 = pl.pallas_call(
    _tc2_body,
    grid=(NB,),
    in_specs=[_p_spec, _row_spec, _deg_spec, _b_spec, _w_spec],
    out_specs=[_row_spec, _row_spec],
    out_shape=[
        jax.ShapeDtypeStruct((N, D), jnp.float32),
        jax.ShapeDtypeStruct((N, D), jnp.float32),
    ],
)

_tc3 = pl.pallas_call(
    _tc3_body,
    grid=(NB,),
    in_specs=[
        _p_spec, _row_spec, _deg_spec, _b_spec,
        pl.BlockSpec((D, 1), lambda i: (0, 0)),
        pl.BlockSpec((1, 1), lambda i: (0, 0)),
    ],
    out_specs=pl.BlockSpec((BN, 1), lambda i: (i, 0)),
    out_shape=jax.ShapeDtypeStruct((N, 1), jnp.float32),
)


def kernel(x, edge_index, W1, b1, W2, b2, W_out, b_out):
    src = edge_index[0].reshape(NW, NBLKA, BCA, KA)
    dst = edge_index[1].reshape(NW, NBLKA, BCA, KA)
    dst_deg = edge_index[1].reshape(NW, NCHD, KD)
    z128 = jnp.zeros((NPAD, D), jnp.float32)
    z16 = jnp.zeros((NPAD, DEG_W), jnp.float32)

    degp = _deg_kernel(dst_deg, z16)
    h1 = _tc1a(x, W1)
    h1p = _tc1b(degp, h1)
    p1 = _agg_kernel(h1p, src, dst, z128)
    h2, h2p = _tc2(p1, h1, degp, b1.reshape(1, D), W2)
    p2 = _agg_kernel(h2p, src, dst, z128)
    out = _tc3(p2, h2, degp, b2.reshape(1, D), W_out, b_out.reshape(1, 1))
    return out


# speedup vs baseline: 3.4417x; 1.0008x over previous
import functools

import jax
import jax.numpy as jnp
from jax import lax
from jax.experimental import pallas as pl
from jax.experimental.pallas import tpu as pltpu
from jax.experimental.pallas import tpu_sc as plsc

N = 10000
E = 320000
D = 128

NC = 2
NS = 16
NW = NC * NS
EPT = E // NW
K = 100
NCH = EPT // K
NPAIR = NCH // 2
NPAD = 10240
RPT = NPAD // NS
DEG_W = 16

_mesh = plsc.VectorSubcoreMesh(core_axis_name="c", subcore_axis_name="s")


KD = 125
NCHD = EPT // KD
DEG_WIN = 2


@functools.partial(
    pl.kernel,
    out_type=jax.ShapeDtypeStruct((NC, NPAD, DEG_W), jnp.float32),
    mesh=_mesh,
    scratch_types=[
        pltpu.VMEM((NCHD, KD), jnp.int32),
        pltpu.VMEM((KD, DEG_W), jnp.float32),
        pltpu.SemaphoreType.DMA,
        pltpu.VMEM_SHARED((NPAD, DEG_W), jnp.float32),
    ],
)
def _deg_kernel(dst_hbm, z_hbm, out_hbm, dst_v, ones_v, sem, deg_sh):
    c_id = lax.axis_index("c")
    s_id = lax.axis_index("s")
    wid = c_id * NS + s_id

    pltpu.sync_copy(dst_hbm.at[wid], dst_v)

    @pl.when(s_id == 0)
    def _():
        pltpu.sync_copy(z_hbm, deg_sh)

    def fill(i, carry):
        ones_v[i] = jnp.ones((DEG_W,), jnp.float32)
        return carry

    lax.fori_loop(0, KD, fill, 0)
    plsc.subcore_barrier()

    def step(i, carry):
        pltpu.async_copy(ones_v, deg_sh.at[dst_v.at[i]], sem, add=True)

        @pl.when(i >= DEG_WIN)
        def _():
            pltpu.make_async_copy(ones_v, deg_sh.at[dst_v.at[0]], sem).wait()

        return carry

    lax.fori_loop(0, NCHD, step, 0)

    def drain(i, carry):
        pltpu.make_async_copy(ones_v, deg_sh.at[dst_v.at[0]], sem).wait()
        return carry

    lax.fori_loop(0, DEG_WIN, drain, 0)
    plsc.subcore_barrier()

    pltpu.sync_copy(
        deg_sh.at[pl.ds(s_id * RPT, RPT)],
        out_hbm.at[c_id, pl.ds(s_id * RPT, RPT)],
    )


KA = 125
NCHA = EPT // KA
NBLKA = 4
BCA = NCHA // NBLKA
PAIRA = BCA // 2


@functools.partial(
    pl.kernel,
    out_type=jax.ShapeDtypeStruct((NC, NPAD, D), jnp.float32),
    mesh=_mesh,
    scratch_types=[
        pltpu.VMEM((BCA, KA), jnp.int32),
        pltpu.VMEM((BCA, KA), jnp.int32),
        pltpu.VMEM((BCA, KA), jnp.int32),
        pltpu.VMEM((BCA, KA), jnp.int32),
        pltpu.VMEM((KA, D), jnp.float32),
        pltpu.VMEM((KA, D), jnp.float32),
        pltpu.SemaphoreType.DMA,
        pltpu.SemaphoreType.DMA,
        pltpu.SemaphoreType.DMA,
        pltpu.SemaphoreType.DMA,
        pltpu.SemaphoreType.DMA,
        pltpu.VMEM_SHARED((NPAD, D), jnp.float32),
    ],
)
def _agg_kernel(h_hbm, src_hbm, dst_hbm, z_hbm, out_hbm,
                src_b0, src_b1, dst_b0, dst_b1, rows0, rows1,
                gsem0, gsem1, ssem0, ssem1, isem, agg_sh):
    src_bufs = (src_b0, src_b1)
    dst_bufs = (dst_b0, dst_b1)
    ring = ((rows0, gsem0, ssem0), (rows1, gsem1, ssem1))
    c_id = lax.axis_index("c")
    s_id = lax.axis_index("s")
    wid = c_id * NS + s_id

    pltpu.sync_copy(src_hbm.at[wid, 0], src_b0)
    pltpu.sync_copy(dst_hbm.at[wid, 0], dst_b0)

    @pl.when(s_id == 0)
    def _():
        pltpu.sync_copy(z_hbm, agg_sh)

    plsc.subcore_barrier()

    for j, (rb, gs, _) in enumerate(ring):
        pltpu.async_copy(h_hbm.at[src_b0.at[j]], rb, gs)

    for b in range(NBLKA):
        sv = src_bufs[b % 2]
        dv = dst_bufs[b % 2]
        svn = src_bufs[(b + 1) % 2]
        dvn = dst_bufs[(b + 1) % 2]
        if b + 1 < NBLKA:
            pltpu.async_copy(src_hbm.at[wid, b + 1], svn, isem)
            pltpu.async_copy(dst_hbm.at[wid, b + 1], dvn, isem)

        def pair(t, carry):
            base = 2 * t
            for j, (rb, gs, ss) in enumerate(ring):
                pltpu.make_async_copy(h_hbm.at[sv.at[base + j]], rb, gs).wait()
                pltpu.async_copy(rb, agg_sh.at[dv.at[base + j]], ss, add=True)

            @pl.when(t + 1 < PAIRA)
            def _():
                for j, (rb, gs, ss) in enumerate(ring):
                    pltpu.make_async_copy(rb, agg_sh.at[dv.at[base + j]], ss).wait()
                    pltpu.async_copy(h_hbm.at[sv.at[base + 2 + j]], rb, gs)

            return carry

        lax.fori_loop(0, PAIRA, pair, 0)

        if b + 1 < NBLKA:
            pltpu.make_async_copy(src_hbm.at[wid, b + 1], svn, isem).wait()
            pltpu.make_async_copy(dst_hbm.at[wid, b + 1], dvn, isem).wait()
            for j, (rb, gs, ss) in enumerate(ring):
                pltpu.make_async_copy(rb, agg_sh.at[dv.at[BCA - 2 + j]], ss).wait()
                pltpu.async_copy(h_hbm.at[svn.at[j]], rb, gs)
        else:
            for j, (rb, gs, ss) in enumerate(ring):
                pltpu.make_async_copy(rb, agg_sh.at[dv.at[BCA - 2 + j]], ss).wait()

    plsc.subcore_barrier()

    pltpu.sync_copy(
        agg_sh.at[pl.ds(s_id * RPT, RPT)],
        out_hbm.at[c_id, pl.ds(s_id * RPT, RPT)],
    )


BN = 1000
NB = N // BN


def _dinv_of(deg_ref):
    deg = deg_ref[0][:, 0:1] + deg_ref[1][:, 0:1] + 1.0
    return lax.rsqrt(deg)


def _tc1_body(deg_ref, x_ref, xp_ref):
    xp_ref[...] = x_ref[...] * _dinv_of(deg_ref)


def _tc2_body(p_ref, x_ref, deg_ref, b_ref, w1_ref, w2_ref, h2_ref, h2p_ref):
    dinv = _dinv_of(deg_ref)
    a = p_ref[0] + p_ref[1]
    u = a * dinv + x_ref[...] * (dinv * dinv)
    y = jnp.dot(u, w1_ref[...], preferred_element_type=jnp.float32)
    y = jnp.maximum(y + b_ref[...], 0.0)
    h2 = jnp.dot(y, w2_ref[...], preferred_element_type=jnp.float32)
    h2_ref[...] = h2
    h2p_ref[...] = h2 * dinv


def _tc3_body(p_ref, h2_ref, deg_ref, b_ref, wo_ref, bo_ref, out_ref):
    dinv = _dinv_of(deg_ref)
    a = p_ref[0] + p_ref[1]
    y = a * dinv + h2_ref[...] * (dinv * dinv) + b_ref[...]
    out_ref[...] = (
        jnp.dot(y, wo_ref[...], preferred_element_type=jnp.float32)
        + bo_ref[...]
    )


_deg_spec = pl.BlockSpec((2, BN, DEG_W), lambda i: (0, i, 0))
_row_spec = pl.BlockSpec((BN, D), lambda i: (i, 0))
_p_spec = pl.BlockSpec((2, BN, D), lambda i: (0, i, 0))
_w_spec = pl.BlockSpec((D, D), lambda i: (0, 0))
_b_spec = pl.BlockSpec((1, D), lambda i: (0, 0))

_tc1 = pl.pallas_call(
    _tc1_body,
    grid=(NB,),
    in_specs=[_deg_spec, _row_spec],
    out_specs=_row_spec,
    out_shape=jax.ShapeDtypeStruct((N, D), jnp.float32),
)

_tc2 = pl.pallas_call(
    _tc2_body,
    grid=(NB,),
    in_specs=[_p_spec, _row_spec, _deg_spec, _b_spec, _w_spec, _w_spec],
    out_specs=[_row_spec, _row_spec],
    out_shape=[
        jax.ShapeDtypeStruct((N, D), jnp.float32),
        jax.ShapeDtypeStruct((N, D), jnp.float32),
    ],
)

_tc3 = pl.pallas_call(
    _tc3_body,
    grid=(NB,),
    in_specs=[
        _p_spec, _row_spec, _deg_spec, _b_spec,
        pl.BlockSpec((D, 1), lambda i: (0, 0)),
        pl.BlockSpec((1, 1), lambda i: (0, 0)),
    ],
    out_specs=pl.BlockSpec((BN, 1), lambda i: (i, 0)),
    out_shape=jax.ShapeDtypeStruct((N, 1), jnp.float32),
)


def kernel(x, edge_index, W1, b1, W2, b2, W_out, b_out):
    src = edge_index[0].reshape(NW, NBLKA, BCA, KA)
    dst = edge_index[1].reshape(NW, NBLKA, BCA, KA)
    dst_deg = edge_index[1].reshape(NW, NCHD, KD)
    z128 = jnp.zeros((NPAD, D), jnp.float32)
    z16 = jnp.zeros((NPAD, DEG_W), jnp.float32)

    degp = _deg_kernel(dst_deg, z16)
    xp = _tc1(degp, x)
    p1 = _agg_kernel(xp, src, dst, z128)
    h2, h2p = _tc2(p1, x, degp, b1.reshape(1, D), W1, W2)
    p2 = _agg_kernel(h2p, src, dst, z128)
    out = _tc3(p2, h2, degp, b2.reshape(1, D), W_out, b_out.reshape(1, 1))
    return out
